# baseline (device time: 49409 ns/iter reference)
import jax
import jax.numpy as jnp
from jax import lax
from jax.experimental import pallas as pl
from jax.experimental.pallas import tpu as pltpu

N_DEV = 4
B, SQ, SKV, HQ, DH = 2, 128, 512, 4, 64
D_MODEL = 512
SKV_SH = SKV // N_DEV
BH = B * HQ


def _body(x_ref, wq_ref, k_ref, v_ref, wo_ref, out_ref,
          k_buf, v_buf, ks_sems, kr_sems, vs_sems, vr_sems):
    my = lax.axis_index("i")
    left = lax.rem(my + N_DEV - 1, N_DEV)
    right = lax.rem(my + 1, N_DEV)

    barrier = pltpu.get_barrier_semaphore()
    for nbr in (left, right):
        pl.semaphore_signal(barrier, inc=1, device_id=(nbr,),
                            device_id_type=pl.DeviceIdType.MESH)
    pl.semaphore_wait(barrier, 2)

    k_buf[pl.ds(my, 1)] = k_ref[...][None]
    v_buf[pl.ds(my, 1)] = v_ref[...][None]

    for h in range(N_DEV - 1):
        slot = lax.rem(my - h + N_DEV, N_DEV)
        k_rdma = pltpu.make_async_remote_copy(
            src_ref=k_buf.at[slot], dst_ref=k_buf.at[slot],
            send_sem=ks_sems.at[h], recv_sem=kr_sems.at[h],
            device_id=(right,), device_id_type=pl.DeviceIdType.MESH)
        v_rdma = pltpu.make_async_remote_copy(
            src_ref=v_buf.at[slot], dst_ref=v_buf.at[slot],
            send_sem=vs_sems.at[h], recv_sem=vr_sems.at[h],
            device_id=(right,), device_id_type=pl.DeviceIdType.MESH)
        k_rdma.start()
        v_rdma.start()
        k_rdma.wait()
        v_rdma.wait()

    q = jnp.dot(x_ref[...], wq_ref[...],
                preferred_element_type=jnp.float32)
    qi = lax.broadcasted_iota(jnp.int32, (SQ, SKV_SH), 0)
    kj = lax.broadcasted_iota(jnp.int32, (SQ, SKV_SH), 1)

    ctx_rows = []
    for b in range(B):
        heads = []
        for h in range(HQ):
            bh = b * HQ + h
            qbh = q[b * SQ:(b + 1) * SQ, h * DH:(h + 1) * DH]
            s_chunks = []
            for c in range(N_DEV):
                kc = k_buf[c, bh]
                s = lax.dot_general(
                    qbh, kc, (((1,), (1,)), ((), ())),
                    preferred_element_type=jnp.float32) * 0.125
                kg = kj + c * SKV_SH
                mask = (jnp.abs(qi - kg) <= 128) | (kg < 32) | (qi < 32)
                s_chunks.append(jnp.where(mask, s, -1e9))
            s_all = jnp.concatenate(s_chunks, axis=1)
            m = jnp.max(s_all, axis=1, keepdims=True)
            w = jnp.exp(s_all - m)
            p = w / jnp.sum(w, axis=1, keepdims=True)
            ctx = sum(
                jnp.dot(p[:, c * SKV_SH:(c + 1) * SKV_SH], v_buf[c, bh],
                        preferred_element_type=jnp.float32)
                for c in range(N_DEV))
            heads.append(ctx)
        ctx_rows.append(jnp.concatenate(heads, axis=1))
    ctx_all = jnp.concatenate(ctx_rows, axis=0)
    out_ref[...] = jnp.dot(ctx_all, wo_ref[...],
                           preferred_element_type=jnp.float32)


def kernel(x, Wq, K_ext, V_ext, Wo):
    x2 = x.reshape(B * SQ, D_MODEL)
    k3 = K_ext.transpose(0, 2, 1, 3).reshape(BH, SKV_SH, DH)
    v3 = V_ext.transpose(0, 2, 1, 3).reshape(BH, SKV_SH, DH)

    out2 = pl.pallas_call(
        _body,
        out_shape=jax.ShapeDtypeStruct((B * SQ, D_MODEL), jnp.float32),
        in_specs=[pl.BlockSpec(memory_space=pltpu.VMEM)] * 5,
        out_specs=pl.BlockSpec(memory_space=pltpu.VMEM),
        scratch_shapes=[
            pltpu.VMEM((N_DEV, BH, SKV_SH, DH), jnp.float32),
            pltpu.VMEM((N_DEV, BH, SKV_SH, DH), jnp.float32),
            pltpu.SemaphoreType.DMA((N_DEV - 1,)),
            pltpu.SemaphoreType.DMA((N_DEV - 1,)),
            pltpu.SemaphoreType.DMA((N_DEV - 1,)),
            pltpu.SemaphoreType.DMA((N_DEV - 1,)),
        ],
        compiler_params=pltpu.CompilerParams(collective_id=0),
    )(x2, Wq, k3, v3, Wo)
    return out2.reshape(B, SQ, D_MODEL)


# device time: 24031 ns/iter; 2.0561x vs baseline; 2.0561x over previous
import jax
import jax.numpy as jnp
from jax import lax
from jax.experimental import pallas as pl
from jax.experimental.pallas import tpu as pltpu

N_DEV = 4
B, SQ, SKV, HQ, DH = 2, 128, 512, 4, 64
D_MODEL = 512
SKV_SH = SKV // N_DEV
BH = B * HQ


def _body(x_ref, wq_ref, k_ref, v_ref, wo_ref, out_ref,
          ctx_buf, stat_buf, cs_sems, cr_sems, ss_sems, sr_sems):
    my = lax.axis_index("i")

    barrier = pltpu.get_barrier_semaphore()
    for d in range(1, N_DEV):
        peer = lax.rem(my + d, N_DEV)
        pl.semaphore_signal(barrier, inc=1, device_id=(peer,),
                            device_id_type=pl.DeviceIdType.MESH)
    pl.semaphore_wait(barrier, N_DEV - 1)

    q = jnp.dot(x_ref[...], wq_ref[...],
                preferred_element_type=jnp.float32)

    qi = lax.broadcasted_iota(jnp.int32, (SQ, SKV_SH), 0)
    kj = lax.broadcasted_iota(jnp.int32, (SQ, SKV_SH), 1) + my * SKV_SH
    mask = (jnp.abs(qi - kj) <= 128) | (kj < 32) | (qi < 32)

    ctx_list = []
    m_list = []
    l_list = []
    for b in range(B):
        for h in range(HQ):
            bh = b * HQ + h
            qbh = q[b * SQ:(b + 1) * SQ, h * DH:(h + 1) * DH]
            s = lax.dot_general(
                qbh, k_ref[bh], (((1,), (1,)), ((), ())),
                preferred_element_type=jnp.float32) * 0.125
            s = jnp.where(mask, s, -1e9)
            m = jnp.max(s, axis=1, keepdims=True)
            w = jnp.exp(s - m)
            l = jnp.sum(w, axis=1, keepdims=True)
            ctx_list.append(jnp.dot(w, v_ref[bh],
                                    preferred_element_type=jnp.float32))
            m_list.append(m)
            l_list.append(l)

    ctx_mine = jnp.stack(ctx_list, axis=0)
    stats_mine = jnp.concatenate(m_list + l_list, axis=1)
    ctx_buf[pl.ds(my, 1)] = ctx_mine[None]
    stat_buf[pl.ds(my, 1)] = stats_mine[None]

    sends = []
    for d in range(1, N_DEV):
        peer = lax.rem(my + d, N_DEV)
        c_rdma = pltpu.make_async_remote_copy(
            src_ref=ctx_buf.at[my], dst_ref=ctx_buf.at[my],
            send_sem=cs_sems.at[d - 1], recv_sem=cr_sems.at[my],
            device_id=(peer,), device_id_type=pl.DeviceIdType.MESH)
        s_rdma = pltpu.make_async_remote_copy(
            src_ref=stat_buf.at[my], dst_ref=stat_buf.at[my],
            send_sem=ss_sems.at[d - 1], recv_sem=sr_sems.at[my],
            device_id=(peer,), device_id_type=pl.DeviceIdType.MESH)
        c_rdma.start()
        s_rdma.start()
        sends.append((c_rdma, s_rdma))

    for d in range(1, N_DEV):
        origin = lax.rem(my + d, N_DEV)
        c_recv = pltpu.make_async_remote_copy(
            src_ref=ctx_buf.at[origin], dst_ref=ctx_buf.at[origin],
            send_sem=cs_sems.at[d - 1], recv_sem=cr_sems.at[origin],
            device_id=(origin,), device_id_type=pl.DeviceIdType.MESH)
        s_recv = pltpu.make_async_remote_copy(
            src_ref=stat_buf.at[origin], dst_ref=stat_buf.at[origin],
            send_sem=ss_sems.at[d - 1], recv_sem=sr_sems.at[origin],
            device_id=(origin,), device_id_type=pl.DeviceIdType.MESH)
        c_recv.wait_recv()
        s_recv.wait_recv()

    stats = [stat_buf[c] for c in range(N_DEV)]
    ms = [st[:, :BH] for st in stats]
    ls = [st[:, BH:] for st in stats]
    m_glob = ms[0]
    for c in range(1, N_DEV):
        m_glob = jnp.maximum(m_glob, ms[c])
    scales = [jnp.exp(ms[c] - m_glob) for c in range(N_DEV)]
    denom = sum(ls[c] * scales[c] for c in range(N_DEV))

    ctx_rows = []
    for b in range(B):
        heads = []
        for h in range(HQ):
            bh = b * HQ + h
            num = sum(ctx_buf[c, bh] * scales[c][:, bh:bh + 1]
                      for c in range(N_DEV))
            heads.append(num / denom[:, bh:bh + 1])
        ctx_rows.append(jnp.concatenate(heads, axis=1))
    ctx_all = jnp.concatenate(ctx_rows, axis=0)
    out_ref[...] = jnp.dot(ctx_all, wo_ref[...],
                           preferred_element_type=jnp.float32)

    for c_rdma, s_rdma in sends:
        c_rdma.wait_send()
        s_rdma.wait_send()


def kernel(x, Wq, K_ext, V_ext, Wo):
    x2 = x.reshape(B * SQ, D_MODEL)
    k3 = K_ext.transpose(0, 2, 1, 3).reshape(BH, SKV_SH, DH)
    v3 = V_ext.transpose(0, 2, 1, 3).reshape(BH, SKV_SH, DH)

    out2 = pl.pallas_call(
        _body,
        out_shape=jax.ShapeDtypeStruct((B * SQ, D_MODEL), jnp.float32),
        in_specs=[pl.BlockSpec(memory_space=pltpu.VMEM)] * 5,
        out_specs=pl.BlockSpec(memory_space=pltpu.VMEM),
        scratch_shapes=[
            pltpu.VMEM((N_DEV, BH, SQ, DH), jnp.float32),
            pltpu.VMEM((N_DEV, SQ, 2 * BH), jnp.float32),
            pltpu.SemaphoreType.DMA((N_DEV - 1,)),
            pltpu.SemaphoreType.DMA((N_DEV,)),
            pltpu.SemaphoreType.DMA((N_DEV - 1,)),
            pltpu.SemaphoreType.DMA((N_DEV,)),
        ],
        compiler_params=pltpu.CompilerParams(collective_id=0),
    )(x2, Wq, k3, v3, Wo)
    return out2.reshape(B, SQ, D_MODEL)


# device time: 22879 ns/iter; 2.1596x vs baseline; 1.0504x over previous
import jax
import jax.numpy as jnp
from jax import lax
from jax.experimental import pallas as pl
from jax.experimental.pallas import tpu as pltpu

N_DEV = 4
B, SQ, SKV, HQ, DH = 2, 128, 512, 4, 64
D_MODEL = 512
SKV_SH = SKV // N_DEV
BH = B * HQ
GR = 32


def _flash_rows(q, k_ref, v_ref, rows, row0, mask):
    ctx_list, m_list, l_list = [], [], []
    for b in range(B):
        for h in range(HQ):
            bh = b * HQ + h
            qbh = q[b * SQ + row0:b * SQ + row0 + rows,
                    h * DH:(h + 1) * DH]
            s = lax.dot_general(
                qbh, k_ref[bh], (((1,), (1,)), ((), ())),
                preferred_element_type=jnp.float32) * 0.125
            if mask is not None:
                s = jnp.where(mask, s, -1e9)
            m = jnp.max(s, axis=1, keepdims=True)
            w = jnp.exp(s - m)
            l = jnp.sum(w, axis=1, keepdims=True)
            ctx_list.append(jnp.dot(w, v_ref[bh],
                                    preferred_element_type=jnp.float32))
            m_list.append(m)
            l_list.append(l)
    ctx = jnp.stack(ctx_list, axis=0)
    stats = jnp.concatenate(m_list + l_list, axis=1)
    return ctx, stats


def _body(x_ref, wq_ref, k_ref, v_ref, wo_ref, out_ref,
          ctx_buf, stat_buf, cs_sems, cr_sems, ss_sems, sr_sems):
    my = lax.axis_index("i")

    barrier = pltpu.get_barrier_semaphore()
    for d in range(1, N_DEV):
        peer = lax.rem(my + d, N_DEV)
        pl.semaphore_signal(barrier, inc=1, device_id=(peer,),
                            device_id_type=pl.DeviceIdType.MESH)
    pl.semaphore_wait(barrier, N_DEV - 1)

    q = jnp.dot(x_ref[...], wq_ref[...],
                preferred_element_type=jnp.float32)

    @pl.when(my == 0)
    def _():
        ctx, stats = _flash_rows(q, k_ref, v_ref, SQ, 0, None)
        ctx_buf[pl.ds(my, 1)] = ctx[None]
        stat_buf[pl.ds(my, 1)] = stats[None]

    @pl.when(my == 1)
    def _():
        qi = lax.broadcasted_iota(jnp.int32, (SQ, SKV_SH), 0)
        kj = lax.broadcasted_iota(jnp.int32, (SQ, SKV_SH), 1)
        mask = (kj <= qi) | (qi < GR)
        ctx, stats = _flash_rows(q, k_ref, v_ref, SQ, 0, mask)
        ctx_buf[pl.ds(my, 1)] = ctx[None]
        stat_buf[pl.ds(my, 1)] = stats[None]

    @pl.when(my >= 2)
    def _():
        ctx, stats = _flash_rows(q, k_ref, v_ref, GR, 0, None)
        ctx_buf[pl.ds(my, 1), :, pl.ds(0, GR), :] = ctx[None]
        stat_buf[pl.ds(my, 1), pl.ds(0, GR), :] = stats[None]

    sends = []
    for d in range(1, N_DEV):
        peer = lax.rem(my + d, N_DEV)

        @pl.when(my < 2)
        def _():
            s_rdma = pltpu.make_async_remote_copy(
                src_ref=stat_buf.at[my], dst_ref=stat_buf.at[my],
                send_sem=ss_sems.at[d - 1], recv_sem=sr_sems.at[my],
                device_id=(peer,), device_id_type=pl.DeviceIdType.MESH)
            c_rdma = pltpu.make_async_remote_copy(
                src_ref=ctx_buf.at[my], dst_ref=ctx_buf.at[my],
                send_sem=cs_sems.at[d - 1], recv_sem=cr_sems.at[my],
                device_id=(peer,), device_id_type=pl.DeviceIdType.MESH)
            s_rdma.start()
            c_rdma.start()

        @pl.when(my >= 2)
        def _():
            s_rdma = pltpu.make_async_remote_copy(
                src_ref=stat_buf.at[my, pl.ds(0, GR), :],
                dst_ref=stat_buf.at[my, pl.ds(0, GR), :],
                send_sem=ss_sems.at[d - 1], recv_sem=sr_sems.at[my],
                device_id=(peer,), device_id_type=pl.DeviceIdType.MESH)
            c_rdma = pltpu.make_async_remote_copy(
                src_ref=ctx_buf.at[my, :, pl.ds(0, GR), :],
                dst_ref=ctx_buf.at[my, :, pl.ds(0, GR), :],
                send_sem=cs_sems.at[d - 1], recv_sem=cr_sems.at[my],
                device_id=(peer,), device_id_type=pl.DeviceIdType.MESH)
            s_rdma.start()
            c_rdma.start()

    for d in range(1, N_DEV):
        origin = lax.rem(my + d, N_DEV)

        @pl.when(origin < 2)
        def _():
            pltpu.make_async_remote_copy(
                src_ref=stat_buf.at[origin], dst_ref=stat_buf.at[origin],
                send_sem=ss_sems.at[d - 1], recv_sem=sr_sems.at[origin],
                device_id=(origin,), device_id_type=pl.DeviceIdType.MESH,
            ).wait_recv()

        @pl.when(origin >= 2)
        def _():
            pltpu.make_async_remote_copy(
                src_ref=stat_buf.at[origin, pl.ds(0, GR), :],
                dst_ref=stat_buf.at[origin, pl.ds(0, GR), :],
                send_sem=ss_sems.at[d - 1], recv_sem=sr_sems.at[origin],
                device_id=(origin,), device_id_type=pl.DeviceIdType.MESH,
            ).wait_recv()

    for d in range(1, N_DEV):
        origin = lax.rem(my + d, N_DEV)

        @pl.when(origin < 2)
        def _():
            pltpu.make_async_remote_copy(
                src_ref=ctx_buf.at[origin], dst_ref=ctx_buf.at[origin],
                send_sem=cs_sems.at[d - 1], recv_sem=cr_sems.at[origin],
                device_id=(origin,), device_id_type=pl.DeviceIdType.MESH,
            ).wait_recv()

        @pl.when(origin >= 2)
        def _():
            pltpu.make_async_remote_copy(
                src_ref=ctx_buf.at[origin, :, pl.ds(0, GR), :],
                dst_ref=ctx_buf.at[origin, :, pl.ds(0, GR), :],
                send_sem=cs_sems.at[d - 1], recv_sem=cr_sems.at[origin],
                device_id=(origin,), device_id_type=pl.DeviceIdType.MESH,
            ).wait_recv()

    stA = [stat_buf[c, :GR] for c in range(N_DEV)]
    stB = [stat_buf[c, GR:] for c in range(2)]
    mA = stA[0][:, :BH]
    for c in range(1, N_DEV):
        mA = jnp.maximum(mA, stA[c][:, :BH])
    sclA = [jnp.exp(stA[c][:, :BH] - mA) for c in range(N_DEV)]
    denA = sum(stA[c][:, BH:] * sclA[c] for c in range(N_DEV))
    mB = jnp.maximum(stB[0][:, :BH], stB[1][:, :BH])
    sclB = [jnp.exp(stB[c][:, :BH] - mB) for c in range(2)]
    denB = sum(stB[c][:, BH:] * sclB[c] for c in range(2))

    ctx_rows = []
    for b in range(B):
        heads = []
        for h in range(HQ):
            bh = b * HQ + h
            numA = sum(ctx_buf[c, bh, :GR] * sclA[c][:, bh:bh + 1]
                       for c in range(N_DEV))
            numB = sum(ctx_buf[c, bh, GR:] * sclB[c][:, bh:bh + 1]
                       for c in range(2))
            heads.append(jnp.concatenate(
                [numA / denA[:, bh:bh + 1], numB / denB[:, bh:bh + 1]],
                axis=0))
        ctx_rows.append(jnp.concatenate(heads, axis=1))
    ctx_all = jnp.concatenate(ctx_rows, axis=0)
    out_ref[...] = jnp.dot(ctx_all, wo_ref[...],
                           preferred_element_type=jnp.float32)

    for d in range(1, N_DEV):
        @pl.when(my < 2)
        def _():
            pltpu.make_async_remote_copy(
                src_ref=stat_buf.at[my], dst_ref=stat_buf.at[my],
                send_sem=ss_sems.at[d - 1], recv_sem=sr_sems.at[my],
                device_id=(my,), device_id_type=pl.DeviceIdType.MESH,
            ).wait_send()
            pltpu.make_async_remote_copy(
                src_ref=ctx_buf.at[my], dst_ref=ctx_buf.at[my],
                send_sem=cs_sems.at[d - 1], recv_sem=cr_sems.at[my],
                device_id=(my,), device_id_type=pl.DeviceIdType.MESH,
            ).wait_send()

        @pl.when(my >= 2)
        def _():
            pltpu.make_async_remote_copy(
                src_ref=stat_buf.at[my, pl.ds(0, GR), :],
                dst_ref=stat_buf.at[my, pl.ds(0, GR), :],
                send_sem=ss_sems.at[d - 1], recv_sem=sr_sems.at[my],
                device_id=(my,), device_id_type=pl.DeviceIdType.MESH,
            ).wait_send()
            pltpu.make_async_remote_copy(
                src_ref=ctx_buf.at[my, :, pl.ds(0, GR), :],
                dst_ref=ctx_buf.at[my, :, pl.ds(0, GR), :],
                send_sem=cs_sems.at[d - 1], recv_sem=cr_sems.at[my],
                device_id=(my,), device_id_type=pl.DeviceIdType.MESH,
            ).wait_send()


def kernel(x, Wq, K_ext, V_ext, Wo):
    x2 = x.reshape(B * SQ, D_MODEL)
    k3 = K_ext.transpose(0, 2, 1, 3).reshape(BH, SKV_SH, DH)
    v3 = V_ext.transpose(0, 2, 1, 3).reshape(BH, SKV_SH, DH)

    out2 = pl.pallas_call(
        _body,
        out_shape=jax.ShapeDtypeStruct((B * SQ, D_MODEL), jnp.float32),
        in_specs=[pl.BlockSpec(memory_space=pltpu.VMEM)] * 5,
        out_specs=pl.BlockSpec(memory_space=pltpu.VMEM),
        scratch_shapes=[
            pltpu.VMEM((N_DEV, BH, SQ, DH), jnp.float32),
            pltpu.VMEM((N_DEV, SQ, 2 * BH), jnp.float32),
            pltpu.SemaphoreType.DMA((N_DEV - 1,)),
            pltpu.SemaphoreType.DMA((N_DEV,)),
            pltpu.SemaphoreType.DMA((N_DEV - 1,)),
            pltpu.SemaphoreType.DMA((N_DEV,)),
        ],
        compiler_params=pltpu.CompilerParams(collective_id=0),
    )(x2, Wq, k3, v3, Wo)
    return out2.reshape(B, SQ, D_MODEL)


# device time: 15996 ns/iter; 3.0888x vs baseline; 1.4303x over previous
import jax
import jax.numpy as jnp
from jax import lax
from jax.experimental import pallas as pl
from jax.experimental.pallas import tpu as pltpu

N_DEV = 4
B, SQ, SKV, HQ, DH = 2, 128, 512, 4, 64
D_MODEL = 512
SKV_SH = SKV // N_DEV
BH = B * HQ
GR = 32


def _flash_rows(q, k_ref, v_ref, rows, row0, mask):
    ctx_list, m_list, l_list = [], [], []
    for b in range(B):
        for h in range(HQ):
            bh = b * HQ + h
            qbh = q[b * SQ + row0:b * SQ + row0 + rows,
                    h * DH:(h + 1) * DH]
            s = lax.dot_general(
                qbh, k_ref[bh], (((1,), (1,)), ((), ())),
                preferred_element_type=jnp.float32) * 0.125
            if mask is not None:
                s = jnp.where(mask, s, -1e9)
            m = jnp.max(s, axis=1, keepdims=True)
            w = jnp.exp(s - m)
            l = jnp.sum(w, axis=1, keepdims=True)
            ctx_list.append(jnp.dot(w, v_ref[bh],
                                    preferred_element_type=jnp.float32))
            m_list.append(m)
            l_list.append(l)
    ctx = jnp.stack(ctx_list, axis=0)
    stats = jnp.concatenate(m_list + l_list, axis=1)
    return ctx, stats


def _body(x_ref, wq_ref, k_ref, v_ref, wo_ref, out_ref,
          ctx_buf, stat_buf, cs_sems, cr_sems, ss_sems, sr_sems):
    my = lax.axis_index("i")

    barrier = pltpu.get_barrier_semaphore()
    for d in range(1, N_DEV):
        peer = lax.rem(my + d, N_DEV)
        pl.semaphore_signal(barrier, inc=1, device_id=(peer,),
                            device_id_type=pl.DeviceIdType.MESH)

    q = jnp.dot(x_ref[...], wq_ref[...],
                preferred_element_type=jnp.float32)

    @pl.when(my == 0)
    def _():
        ctx, stats = _flash_rows(q, k_ref, v_ref, SQ, 0, None)
        ctx_buf[pl.ds(my, 1)] = ctx.astype(jnp.bfloat16)[None]
        stat_buf[pl.ds(my, 1)] = stats[None]

    @pl.when(my == 1)
    def _():
        qi = lax.broadcasted_iota(jnp.int32, (SQ, SKV_SH), 0)
        kj = lax.broadcasted_iota(jnp.int32, (SQ, SKV_SH), 1)
        mask = (kj <= qi) | (qi < GR)
        ctx, stats = _flash_rows(q, k_ref, v_ref, SQ, 0, mask)
        ctx_buf[pl.ds(my, 1)] = ctx.astype(jnp.bfloat16)[None]
        stat_buf[pl.ds(my, 1)] = stats[None]

    @pl.when(my >= 2)
    def _():
        ctx, stats = _flash_rows(q, k_ref, v_ref, GR, 0, None)
        ctx_buf[pl.ds(my, 1), :, pl.ds(0, GR), :] = ctx.astype(jnp.bfloat16)[None]
        stat_buf[pl.ds(my, 1), pl.ds(0, GR), :] = stats[None]

    pl.semaphore_wait(barrier, N_DEV - 1)
    sends = []
    for d in range(1, N_DEV):
        peer = lax.rem(my + d, N_DEV)

        @pl.when(my < 2)
        def _():
            s_rdma = pltpu.make_async_remote_copy(
                src_ref=stat_buf.at[my], dst_ref=stat_buf.at[my],
                send_sem=ss_sems.at[d - 1], recv_sem=sr_sems.at[my],
                device_id=(peer,), device_id_type=pl.DeviceIdType.MESH)
            c_rdma = pltpu.make_async_remote_copy(
                src_ref=ctx_buf.at[my], dst_ref=ctx_buf.at[my],
                send_sem=cs_sems.at[d - 1], recv_sem=cr_sems.at[my],
                device_id=(peer,), device_id_type=pl.DeviceIdType.MESH)
            s_rdma.start()
            c_rdma.start()

        @pl.when(my >= 2)
        def _():
            s_rdma = pltpu.make_async_remote_copy(
                src_ref=stat_buf.at[my, pl.ds(0, GR), :],
                dst_ref=stat_buf.at[my, pl.ds(0, GR), :],
                send_sem=ss_sems.at[d - 1], recv_sem=sr_sems.at[my],
                device_id=(peer,), device_id_type=pl.DeviceIdType.MESH)
            c_rdma = pltpu.make_async_remote_copy(
                src_ref=ctx_buf.at[my, :, pl.ds(0, GR), :],
                dst_ref=ctx_buf.at[my, :, pl.ds(0, GR), :],
                send_sem=cs_sems.at[d - 1], recv_sem=cr_sems.at[my],
                device_id=(peer,), device_id_type=pl.DeviceIdType.MESH)
            s_rdma.start()
            c_rdma.start()

    for d in range(1, N_DEV):
        origin = lax.rem(my + d, N_DEV)

        @pl.when(origin < 2)
        def _():
            pltpu.make_async_remote_copy(
                src_ref=stat_buf.at[origin], dst_ref=stat_buf.at[origin],
                send_sem=ss_sems.at[d - 1], recv_sem=sr_sems.at[origin],
                device_id=(origin,), device_id_type=pl.DeviceIdType.MESH,
            ).wait_recv()

        @pl.when(origin >= 2)
        def _():
            pltpu.make_async_remote_copy(
                src_ref=stat_buf.at[origin, pl.ds(0, GR), :],
                dst_ref=stat_buf.at[origin, pl.ds(0, GR), :],
                send_sem=ss_sems.at[d - 1], recv_sem=sr_sems.at[origin],
                device_id=(origin,), device_id_type=pl.DeviceIdType.MESH,
            ).wait_recv()

    for d in range(1, N_DEV):
        origin = lax.rem(my + d, N_DEV)

        @pl.when(origin < 2)
        def _():
            pltpu.make_async_remote_copy(
                src_ref=ctx_buf.at[origin], dst_ref=ctx_buf.at[origin],
                send_sem=cs_sems.at[d - 1], recv_sem=cr_sems.at[origin],
                device_id=(origin,), device_id_type=pl.DeviceIdType.MESH,
            ).wait_recv()

        @pl.when(origin >= 2)
        def _():
            pltpu.make_async_remote_copy(
                src_ref=ctx_buf.at[origin, :, pl.ds(0, GR), :],
                dst_ref=ctx_buf.at[origin, :, pl.ds(0, GR), :],
                send_sem=cs_sems.at[d - 1], recv_sem=cr_sems.at[origin],
                device_id=(origin,), device_id_type=pl.DeviceIdType.MESH,
            ).wait_recv()

    stA = [stat_buf[c, :GR] for c in range(N_DEV)]
    stB = [stat_buf[c, GR:] for c in range(2)]
    mA = stA[0][:, :BH]
    for c in range(1, N_DEV):
        mA = jnp.maximum(mA, stA[c][:, :BH])
    sclA = [jnp.exp(stA[c][:, :BH] - mA) for c in range(N_DEV)]
    denA = sum(stA[c][:, BH:] * sclA[c] for c in range(N_DEV))
    mB = jnp.maximum(stB[0][:, :BH], stB[1][:, :BH])
    sclB = [jnp.exp(stB[c][:, :BH] - mB) for c in range(2)]
    denB = sum(stB[c][:, BH:] * sclB[c] for c in range(2))

    ctx_rows = []
    for b in range(B):
        heads = []
        for h in range(HQ):
            bh = b * HQ + h
            numA = sum(ctx_buf[c, bh, :GR].astype(jnp.float32) * sclA[c][:, bh:bh + 1]
                       for c in range(N_DEV))
            numB = sum(ctx_buf[c, bh, GR:].astype(jnp.float32) * sclB[c][:, bh:bh + 1]
                       for c in range(2))
            heads.append(jnp.concatenate(
                [numA / denA[:, bh:bh + 1], numB / denB[:, bh:bh + 1]],
                axis=0))
        ctx_rows.append(jnp.concatenate(heads, axis=1))
    ctx_all = jnp.concatenate(ctx_rows, axis=0)
    out_ref[...] = jnp.dot(ctx_all, wo_ref[...],
                           preferred_element_type=jnp.float32)

    for d in range(1, N_DEV):
        @pl.when(my < 2)
        def _():
            pltpu.make_async_remote_copy(
                src_ref=stat_buf.at[my], dst_ref=stat_buf.at[my],
                send_sem=ss_sems.at[d - 1], recv_sem=sr_sems.at[my],
                device_id=(my,), device_id_type=pl.DeviceIdType.MESH,
            ).wait_send()
            pltpu.make_async_remote_copy(
                src_ref=ctx_buf.at[my], dst_ref=ctx_buf.at[my],
                send_sem=cs_sems.at[d - 1], recv_sem=cr_sems.at[my],
                device_id=(my,), device_id_type=pl.DeviceIdType.MESH,
            ).wait_send()

        @pl.when(my >= 2)
        def _():
            pltpu.make_async_remote_copy(
                src_ref=stat_buf.at[my, pl.ds(0, GR), :],
                dst_ref=stat_buf.at[my, pl.ds(0, GR), :],
                send_sem=ss_sems.at[d - 1], recv_sem=sr_sems.at[my],
                device_id=(my,), device_id_type=pl.DeviceIdType.MESH,
            ).wait_send()
            pltpu.make_async_remote_copy(
                src_ref=ctx_buf.at[my, :, pl.ds(0, GR), :],
                dst_ref=ctx_buf.at[my, :, pl.ds(0, GR), :],
                send_sem=cs_sems.at[d - 1], recv_sem=cr_sems.at[my],
                device_id=(my,), device_id_type=pl.DeviceIdType.MESH,
            ).wait_send()


def kernel(x, Wq, K_ext, V_ext, Wo):
    x2 = x.reshape(B * SQ, D_MODEL)
    k3 = K_ext.transpose(0, 2, 1, 3).reshape(BH, SKV_SH, DH)
    v3 = V_ext.transpose(0, 2, 1, 3).reshape(BH, SKV_SH, DH)

    out2 = pl.pallas_call(
        _body,
        out_shape=jax.ShapeDtypeStruct((B * SQ, D_MODEL), jnp.float32),
        in_specs=[pl.BlockSpec(memory_space=pltpu.VMEM)] * 5,
        out_specs=pl.BlockSpec(memory_space=pltpu.VMEM),
        scratch_shapes=[
            pltpu.VMEM((N_DEV, BH, SQ, DH), jnp.bfloat16),
            pltpu.VMEM((N_DEV, SQ, 2 * BH), jnp.float32),
            pltpu.SemaphoreType.DMA((N_DEV - 1,)),
            pltpu.SemaphoreType.DMA((N_DEV,)),
            pltpu.SemaphoreType.DMA((N_DEV - 1,)),
            pltpu.SemaphoreType.DMA((N_DEV,)),
        ],
        compiler_params=pltpu.CompilerParams(collective_id=0),
    )(x2, Wq, k3, v3, Wo)
    return out2.reshape(B, SQ, D_MODEL)


# device time: 15179 ns/iter; 3.2551x vs baseline; 1.0538x over previous
import jax
import jax.numpy as jnp
from jax import lax
from jax.experimental import pallas as pl
from jax.experimental.pallas import tpu as pltpu

N_DEV = 4
B, SQ, SKV, HQ, DH = 2, 128, 512, 4, 64
D_MODEL = 512
SKV_SH = SKV // N_DEV
BH = B * HQ
GR = 32


def _flash_rows(q, k_ref, v_ref, rows, mask):
    ctx_list, m_list, l_list = [], [], []
    for b in range(B):
        for h in range(HQ):
            bh = b * HQ + h
            qbh = q[b * SQ:b * SQ + rows, h * DH:(h + 1) * DH]
            s = lax.dot_general(
                qbh, k_ref[bh], (((1,), (1,)), ((), ())),
                preferred_element_type=jnp.float32) * 0.125
            if mask is not None:
                s = jnp.where(mask, s, -1e9)
            m = jnp.max(s, axis=1, keepdims=True)
            w = jnp.exp(s - m)
            l = jnp.sum(w, axis=1, keepdims=True)
            ctx_list.append(jnp.dot(w, v_ref[bh],
                                    preferred_element_type=jnp.float32))
            m_list.append(m)
            l_list.append(l)
    ctx = jnp.stack(ctx_list, axis=0)
    stats = jnp.concatenate(
        m_list + l_list + [jnp.zeros((rows, DH - 2 * BH), jnp.float32)],
        axis=1)
    payload = jnp.concatenate([ctx, stats[None]], axis=0)
    return payload.astype(jnp.bfloat16)


def _body(x_ref, wq_ref, k_ref, v_ref, wo_ref, out_ref,
          buf, send_sems, recv_sems):
    my = lax.axis_index("i")

    barrier = pltpu.get_barrier_semaphore()
    for d in range(1, N_DEV):
        peer = lax.rem(my + d, N_DEV)
        pl.semaphore_signal(barrier, inc=1, device_id=(peer,),
                            device_id_type=pl.DeviceIdType.MESH)

    q = jnp.dot(x_ref[...], wq_ref[...],
                preferred_element_type=jnp.float32)

    @pl.when(my == 0)
    def _():
        buf[pl.ds(my, 1)] = _flash_rows(q, k_ref, v_ref, SQ, None)[None]

    @pl.when(my == 1)
    def _():
        qi = lax.broadcasted_iota(jnp.int32, (SQ, SKV_SH), 0)
        kj = lax.broadcasted_iota(jnp.int32, (SQ, SKV_SH), 1)
        mask = (kj <= qi) | (qi < GR)
        buf[pl.ds(my, 1)] = _flash_rows(q, k_ref, v_ref, SQ, mask)[None]

    @pl.when(my >= 2)
    def _():
        buf[pl.ds(my, 1), :, pl.ds(0, GR), :] = (
            _flash_rows(q, k_ref, v_ref, GR, None)[None])

    pl.semaphore_wait(barrier, N_DEV - 1)
    for d in range(1, N_DEV):
        peer = lax.rem(my + d, N_DEV)

        @pl.when(my < 2)
        def _():
            pltpu.make_async_remote_copy(
                src_ref=buf.at[my], dst_ref=buf.at[my],
                send_sem=send_sems.at[d - 1], recv_sem=recv_sems.at[my],
                device_id=(peer,), device_id_type=pl.DeviceIdType.MESH,
            ).start()

        @pl.when(my >= 2)
        def _():
            pltpu.make_async_remote_copy(
                src_ref=buf.at[my, :, pl.ds(0, GR), :],
                dst_ref=buf.at[my, :, pl.ds(0, GR), :],
                send_sem=send_sems.at[d - 1], recv_sem=recv_sems.at[my],
                device_id=(peer,), device_id_type=pl.DeviceIdType.MESH,
            ).start()

    for d in range(1, N_DEV):
        origin = lax.rem(my + d, N_DEV)

        @pl.when(origin < 2)
        def _():
            pltpu.make_async_remote_copy(
                src_ref=buf.at[origin], dst_ref=buf.at[origin],
                send_sem=send_sems.at[d - 1], recv_sem=recv_sems.at[origin],
                device_id=(origin,), device_id_type=pl.DeviceIdType.MESH,
            ).wait_recv()

        @pl.when(origin >= 2)
        def _():
            pltpu.make_async_remote_copy(
                src_ref=buf.at[origin, :, pl.ds(0, GR), :],
                dst_ref=buf.at[origin, :, pl.ds(0, GR), :],
                send_sem=send_sems.at[d - 1], recv_sem=recv_sems.at[origin],
                device_id=(origin,), device_id_type=pl.DeviceIdType.MESH,
            ).wait_recv()

    st = [buf[c, BH, :, :2 * BH].astype(jnp.float32) for c in range(N_DEV)]
    stA = [s[:GR] for s in st]
    stB = [st[c][GR:] for c in range(2)]
    mA = stA[0][:, :BH]
    for c in range(1, N_DEV):
        mA = jnp.maximum(mA, stA[c][:, :BH])
    sclA = [jnp.exp(stA[c][:, :BH] - mA) for c in range(N_DEV)]
    denA = sum(stA[c][:, BH:] * sclA[c] for c in range(N_DEV))
    mB = jnp.maximum(stB[0][:, :BH], stB[1][:, :BH])
    sclB = [jnp.exp(stB[c][:, :BH] - mB) for c in range(2)]
    denB = sum(stB[c][:, BH:] * sclB[c] for c in range(2))

    ctx_rows = []
    for b in range(B):
        heads = []
        for h in range(HQ):
            bh = b * HQ + h
            numA = sum(buf[c, bh, :GR].astype(jnp.float32)
                       * sclA[c][:, bh:bh + 1]
                       for c in range(N_DEV))
            numB = sum(buf[c, bh, GR:].astype(jnp.float32)
                       * sclB[c][:, bh:bh + 1]
                       for c in range(2))
            heads.append(jnp.concatenate(
                [numA / denA[:, bh:bh + 1], numB / denB[:, bh:bh + 1]],
                axis=0))
        ctx_rows.append(jnp.concatenate(heads, axis=1))
    ctx_all = jnp.concatenate(ctx_rows, axis=0)
    out_ref[...] = jnp.dot(ctx_all, wo_ref[...],
                           preferred_element_type=jnp.float32)

    for d in range(1, N_DEV):
        @pl.when(my < 2)
        def _():
            pltpu.make_async_remote_copy(
                src_ref=buf.at[my], dst_ref=buf.at[my],
                send_sem=send_sems.at[d - 1], recv_sem=recv_sems.at[my],
                device_id=(my,), device_id_type=pl.DeviceIdType.MESH,
            ).wait_send()

        @pl.when(my >= 2)
        def _():
            pltpu.make_async_remote_copy(
                src_ref=buf.at[my, :, pl.ds(0, GR), :],
                dst_ref=buf.at[my, :, pl.ds(0, GR), :],
                send_sem=send_sems.at[d - 1], recv_sem=recv_sems.at[my],
                device_id=(my,), device_id_type=pl.DeviceIdType.MESH,
            ).wait_send()


def kernel(x, Wq, K_ext, V_ext, Wo):
    x2 = x.reshape(B * SQ, D_MODEL)
    k3 = K_ext.transpose(0, 2, 1, 3).reshape(BH, SKV_SH, DH)
    v3 = V_ext.transpose(0, 2, 1, 3).reshape(BH, SKV_SH, DH)

    out2 = pl.pallas_call(
        _body,
        out_shape=jax.ShapeDtypeStruct((B * SQ, D_MODEL), jnp.float32),
        in_specs=[pl.BlockSpec(memory_space=pltpu.VMEM)] * 5,
        out_specs=pl.BlockSpec(memory_space=pltpu.VMEM),
        scratch_shapes=[
            pltpu.VMEM((N_DEV, BH + 1, SQ, DH), jnp.bfloat16),
            pltpu.SemaphoreType.DMA((N_DEV - 1,)),
            pltpu.SemaphoreType.DMA((N_DEV,)),
        ],
        compiler_params=pltpu.CompilerParams(collective_id=0),
    )(x2, Wq, k3, v3, Wo)
    return out2.reshape(B, SQ, D_MODEL)


# device time: 14868 ns/iter; 3.3232x vs baseline; 1.0209x over previous
import jax
import jax.numpy as jnp
from jax import lax
from jax.experimental import pallas as pl
from jax.experimental.pallas import tpu as pltpu

N_DEV = 4
B, SQ, SKV, HQ, DH = 2, 128, 512, 4, 64
D_MODEL = 512
SKV_SH = SKV // N_DEV
GR = 32
PPB = HQ + 1
NP = B * PPB


def _batch_planes(q, k_ref, v_ref, b, rows, mask):
    ctx_list, m_list, l_list = [], [], []
    for h in range(HQ):
        bh = b * HQ + h
        qbh = q[b * SQ:b * SQ + rows, h * DH:(h + 1) * DH]
        s = lax.dot_general(
            qbh, k_ref[bh], (((1,), (1,)), ((), ())),
            preferred_element_type=jnp.float32) * 0.125
        if mask is not None:
            s = jnp.where(mask, s, -1e9)
        m = jnp.max(s, axis=1, keepdims=True)
        w = jnp.exp(s - m)
        l = jnp.sum(w, axis=1, keepdims=True)
        ctx_list.append(jnp.dot(w, v_ref[bh],
                                preferred_element_type=jnp.float32))
        m_list.append(m)
        l_list.append(l)
    stats = jnp.concatenate(
        m_list + l_list + [jnp.zeros((rows, DH - 2 * HQ), jnp.float32)],
        axis=1)
    payload = jnp.concatenate([jnp.stack(ctx_list, axis=0), stats[None]],
                              axis=0)
    return payload.astype(jnp.bfloat16)


def _body(x_ref, wq_ref, k_ref, v_ref, wo_ref, out_ref,
          buf, send_sems, recv_sems):
    my = lax.axis_index("i")
    d_order = (2, 1, 3)

    barrier = pltpu.get_barrier_semaphore()
    for d in range(1, N_DEV):
        peer = lax.rem(my + d, N_DEV)
        pl.semaphore_signal(barrier, inc=1, device_id=(peer,),
                            device_id_type=pl.DeviceIdType.MESH)

    q = jnp.dot(x_ref[...], wq_ref[...],
                preferred_element_type=jnp.float32)

    qi = lax.broadcasted_iota(jnp.int32, (SQ, SKV_SH), 0)
    kj = lax.broadcasted_iota(jnp.int32, (SQ, SKV_SH), 1)
    mask1 = (kj <= qi) | (qi < GR)

    def send_chunk(k, d):
        peer = lax.rem(my + d, N_DEV)
        sidx = (d - 1) * 2 + k

        @pl.when(my < 2)
        def _():
            pltpu.make_async_remote_copy(
                src_ref=buf.at[my, pl.ds(k * PPB, PPB)],
                dst_ref=buf.at[my, pl.ds(k * PPB, PPB)],
                send_sem=send_sems.at[sidx], recv_sem=recv_sems.at[my, k],
                device_id=(peer,), device_id_type=pl.DeviceIdType.MESH,
            ).start()

        @pl.when(my >= 2)
        def _():
            pltpu.make_async_remote_copy(
                src_ref=buf.at[my, pl.ds(k * PPB, PPB), pl.ds(0, GR), :],
                dst_ref=buf.at[my, pl.ds(k * PPB, PPB), pl.ds(0, GR), :],
                send_sem=send_sems.at[sidx], recv_sem=recv_sems.at[my, k],
                device_id=(peer,), device_id_type=pl.DeviceIdType.MESH,
            ).start()

    @pl.when(my == 0)
    def _():
        buf[pl.ds(my, 1), pl.ds(0, PPB)] = (
            _batch_planes(q, k_ref, v_ref, 0, SQ, None)[None])

    @pl.when(my == 1)
    def _():
        buf[pl.ds(my, 1), pl.ds(0, PPB)] = (
            _batch_planes(q, k_ref, v_ref, 0, SQ, mask1)[None])

    @pl.when(my >= 2)
    def _():
        buf[pl.ds(my, 1), pl.ds(0, PPB), pl.ds(0, GR), :] = (
            _batch_planes(q, k_ref, v_ref, 0, GR, None)[None])

    pl.semaphore_wait(barrier, N_DEV - 1)
    for d in d_order:
        send_chunk(0, d)

    @pl.when(my == 0)
    def _():
        buf[pl.ds(my, 1), pl.ds(PPB, PPB)] = (
            _batch_planes(q, k_ref, v_ref, 1, SQ, None)[None])

    @pl.when(my == 1)
    def _():
        buf[pl.ds(my, 1), pl.ds(PPB, PPB)] = (
            _batch_planes(q, k_ref, v_ref, 1, SQ, mask1)[None])

    @pl.when(my >= 2)
    def _():
        buf[pl.ds(my, 1), pl.ds(PPB, PPB), pl.ds(0, GR), :] = (
            _batch_planes(q, k_ref, v_ref, 1, GR, None)[None])

    for d in d_order:
        send_chunk(1, d)

    for d in range(1, N_DEV):
        origin = lax.rem(my + d, N_DEV)
        for k in range(2):
            @pl.when(origin < 2)
            def _():
                pltpu.make_async_remote_copy(
                    src_ref=buf.at[origin, pl.ds(k * PPB, PPB)],
                    dst_ref=buf.at[origin, pl.ds(k * PPB, PPB)],
                    send_sem=send_sems.at[(d - 1) * 2 + k],
                    recv_sem=recv_sems.at[origin, k],
                    device_id=(origin,), device_id_type=pl.DeviceIdType.MESH,
                ).wait_recv()

            @pl.when(origin >= 2)
            def _():
                pltpu.make_async_remote_copy(
                    src_ref=buf.at[origin, pl.ds(k * PPB, PPB),
                                   pl.ds(0, GR), :],
                    dst_ref=buf.at[origin, pl.ds(k * PPB, PPB),
                                   pl.ds(0, GR), :],
                    send_sem=send_sems.at[(d - 1) * 2 + k],
                    recv_sem=recv_sems.at[origin, k],
                    device_id=(origin,), device_id_type=pl.DeviceIdType.MESH,
                ).wait_recv()

    ctx_rows = []
    for b in range(B):
        st = [buf[c, b * PPB + HQ, :, :2 * HQ].astype(jnp.float32)
              for c in range(N_DEV)]
        stA = [s[:GR] for s in st]
        stB = [st[c][GR:] for c in range(2)]
        mA = stA[0][:, :HQ]
        for c in range(1, N_DEV):
            mA = jnp.maximum(mA, stA[c][:, :HQ])
        sclA = [jnp.exp(stA[c][:, :HQ] - mA) for c in range(N_DEV)]
        denA = sum(stA[c][:, HQ:] * sclA[c] for c in range(N_DEV))
        mB = jnp.maximum(stB[0][:, :HQ], stB[1][:, :HQ])
        sclB = [jnp.exp(stB[c][:, :HQ] - mB) for c in range(2)]
        denB = sum(stB[c][:, HQ:] * sclB[c] for c in range(2))

        heads = []
        for h in range(HQ):
            p = b * PPB + h
            numA = sum(buf[c, p, :GR].astype(jnp.float32)
                       * sclA[c][:, h:h + 1]
                       for c in range(N_DEV))
            numB = sum(buf[c, p, GR:].astype(jnp.float32)
                       * sclB[c][:, h:h + 1]
                       for c in range(2))
            heads.append(jnp.concatenate(
                [numA / denA[:, h:h + 1], numB / denB[:, h:h + 1]],
                axis=0))
        ctx_rows.append(jnp.concatenate(heads, axis=1))
    ctx_all = jnp.concatenate(ctx_rows, axis=0)
    out_ref[...] = jnp.dot(ctx_all, wo_ref[...],
                           preferred_element_type=jnp.float32)

    for d in range(1, N_DEV):
        for k in range(2):
            @pl.when(my < 2)
            def _():
                pltpu.make_async_remote_copy(
                    src_ref=buf.at[my, pl.ds(k * PPB, PPB)],
                    dst_ref=buf.at[my, pl.ds(k * PPB, PPB)],
                    send_sem=send_sems.at[(d - 1) * 2 + k],
                    recv_sem=recv_sems.at[my, k],
                    device_id=(my,), device_id_type=pl.DeviceIdType.MESH,
                ).wait_send()

            @pl.when(my >= 2)
            def _():
                pltpu.make_async_remote_copy(
                    src_ref=buf.at[my, pl.ds(k * PPB, PPB),
                                   pl.ds(0, GR), :],
                    dst_ref=buf.at[my, pl.ds(k * PPB, PPB),
                                   pl.ds(0, GR), :],
                    send_sem=send_sems.at[(d - 1) * 2 + k],
                    recv_sem=recv_sems.at[my, k],
                    device_id=(my,), device_id_type=pl.DeviceIdType.MESH,
                ).wait_send()


def kernel(x, Wq, K_ext, V_ext, Wo):
    x2 = x.reshape(B * SQ, D_MODEL)
    k3 = K_ext.transpose(0, 2, 1, 3).reshape(B * HQ, SKV_SH, DH)
    v3 = V_ext.transpose(0, 2, 1, 3).reshape(B * HQ, SKV_SH, DH)

    out2 = pl.pallas_call(
        _body,
        out_shape=jax.ShapeDtypeStruct((B * SQ, D_MODEL), jnp.float32),
        in_specs=[pl.BlockSpec(memory_space=pltpu.VMEM)] * 5,
        out_specs=pl.BlockSpec(memory_space=pltpu.VMEM),
        scratch_shapes=[
            pltpu.VMEM((N_DEV, NP, SQ, DH), jnp.bfloat16),
            pltpu.SemaphoreType.DMA((2 * (N_DEV - 1),)),
            pltpu.SemaphoreType.DMA((N_DEV, 2)),
        ],
        compiler_params=pltpu.CompilerParams(collective_id=0),
    )(x2, Wq, k3, v3, Wo)
    return out2.reshape(B, SQ, D_MODEL)
